# baseline (speedup 1.0000x reference)
import functools

import jax
import jax.numpy as jnp
from jax import lax
from jax.experimental import pallas as pl
from jax.experimental.pallas import tpu as pltpu
from jax.experimental.pallas import tpu_sc as plsc

BATCH = 4
SEQ_LEN = 8192
EMBED_DIM = 1024
NUM_SEGMENTS = 16
LANES = 16

N_TOKENS = BATCH * SEQ_LEN
NUM_CORES = 2
NUM_SUBCORES = 16
NUM_WORKERS = NUM_CORES * NUM_SUBCORES
TOKENS_PER_WORKER = N_TOKENS // NUM_WORKERS
CHUNK = 16
NUM_CHUNKS = TOKENS_PER_WORKER // CHUNK
NBUF = 4
SBUF = 2
LEAD = 2


@jax.jit
def _seg_encode(x2d, seg, table):
    mesh = plsc.VectorSubcoreMesh(core_axis_name="c", subcore_axis_name="s")

    @functools.partial(
        pl.kernel,
        mesh=mesh,
        out_type=jax.ShapeDtypeStruct((N_TOKENS, EMBED_DIM), jnp.float32),
        scratch_types=[
            pltpu.VMEM((NUM_SEGMENTS, EMBED_DIM), jnp.float32),
            pltpu.VMEM((TOKENS_PER_WORKER,), jnp.int32),
            pltpu.VMEM_SHARED(
                (NUM_SUBCORES, SBUF, CHUNK, EMBED_DIM), jnp.float32),
        ]
        + [pltpu.VMEM((CHUNK, EMBED_DIM), jnp.float32)] * NBUF
        + [pltpu.SemaphoreType.DMA] * NBUF
        + [pltpu.SemaphoreType.DMA] * (2 * SBUF),
    )
    def body(x_hbm, seg_hbm, tab_hbm, out_hbm, tab_v, idx_all, xsh,
             *bufs_sems):
        xbufs = bufs_sems[:NBUF]
        isems = bufs_sems[NBUF:2 * NBUF]
        csems = bufs_sems[2 * NBUF:2 * NBUF + SBUF]
        osems = bufs_sems[2 * NBUF + SBUF:2 * NBUF + 2 * SBUF]
        cid = lax.axis_index("c")
        sid = lax.axis_index("s")
        wid = sid * NUM_CORES + cid
        base = wid * TOKENS_PER_WORKER

        pltpu.sync_copy(tab_hbm, tab_v)
        pltpu.sync_copy(seg_hbm.at[pl.ds(base, TOKENS_PER_WORKER)], idx_all)

        def issue_load(b, c):
            pltpu.async_copy(
                x_hbm.at[pl.ds(base + c * CHUNK, CHUNK)], xbufs[b], isems[b])

        def wait_load(b):
            pltpu.make_async_copy(
                x_hbm.at[pl.ds(0, CHUNK)], xbufs[b], isems[b]).wait()

        def issue_crossbar(b, s):
            pltpu.async_copy(xbufs[b], xsh.at[sid, s], csems[s])

        def wait_crossbar(s):
            pltpu.make_async_copy(xbufs[0], xsh.at[sid, s], csems[s]).wait()

        def issue_store(s, c):
            pltpu.async_copy(
                xsh.at[sid, s], out_hbm.at[pl.ds(base + c * CHUNK, CHUNK)],
                osems[s])

        def wait_store(s):
            pltpu.make_async_copy(
                xsh.at[sid, s], out_hbm.at[pl.ds(0, CHUNK)], osems[s]).wait()

        def compute(b, c):
            xbuf = xbufs[b]
            for g in range(CHUNK // LANES):
                segs = idx_all[pl.ds(c * CHUNK + g * LANES, LANES)]
                for t16 in range(LANES):
                    s = segs[t16]
                    t = g * LANES + t16

                    @plsc.parallel_loop(0, EMBED_DIM, LANES, unroll=16)
                    def pos(j):
                        plsc.addupdate(
                            xbuf.at[t, pl.ds(j, LANES)],
                            tab_v[s, pl.ds(j, LANES)],
                        )

        for b in range(LEAD):
            issue_load(b, b)

        def ring(i, carry):
            c0 = NBUF * i
            for b in range(NBUF):
                c = c0 + b
                s = b % SBUF
                sp = (b - 1) % SBUF
                nb = (b + LEAD) % NBUF
                wait_load(b)

                @pl.when(c + LEAD < NUM_CHUNKS)
                def _():
                    issue_load(nb, c + LEAD)

                compute(b, c)

                @pl.when(c >= 1)
                def _():
                    wait_crossbar(sp)
                    issue_store(sp, c - 1)

                @pl.when(c >= SBUF)
                def _():
                    wait_store(s)

                issue_crossbar(b, s)

            return carry

        lax.fori_loop(0, NUM_CHUNKS // NBUF, ring, 0)

        last = (NUM_CHUNKS - 1) % SBUF
        wait_crossbar(last)
        issue_store(last, NUM_CHUNKS - 1)
        for s in range(SBUF):
            wait_store(s)

    return body(x2d, seg, table)


def kernel(x, segment_ids, table):
    x2d = x.reshape(N_TOKENS, EMBED_DIM)
    seg = segment_ids.reshape(N_TOKENS).astype(jnp.int32)
    out = _seg_encode(x2d, seg, table)
    return out.reshape(BATCH, SEQ_LEN, EMBED_DIM)


# speedup vs baseline: 1.3762x; 1.3762x over previous
import functools

import jax
import jax.numpy as jnp
from jax import lax
from jax.experimental import pallas as pl
from jax.experimental.pallas import tpu as pltpu
from jax.experimental.pallas import tpu_sc as plsc

BATCH = 4
SEQ_LEN = 8192
EMBED_DIM = 1024
NUM_SEGMENTS = 16
LANES = 16

N_TOKENS = BATCH * SEQ_LEN
NUM_CORES = 2
NUM_SUBCORES = 16
NUM_WORKERS = NUM_CORES * NUM_SUBCORES
TOKENS_PER_WORKER = N_TOKENS // NUM_WORKERS
CHUNK = 16
NUM_CHUNKS = TOKENS_PER_WORKER // CHUNK
NBUF = 4
SBUF = 2
LEAD = 2


@jax.jit
def _seg_encode(x2d, seg, table):
    mesh = plsc.VectorSubcoreMesh(core_axis_name="c", subcore_axis_name="s")

    @functools.partial(
        pl.kernel,
        mesh=mesh,
        out_type=jax.ShapeDtypeStruct((N_TOKENS, EMBED_DIM), jnp.float32),
        scratch_types=[
            pltpu.VMEM((NUM_SEGMENTS, EMBED_DIM), jnp.float32),
            pltpu.VMEM((TOKENS_PER_WORKER,), jnp.int32),
            pltpu.VMEM_SHARED(
                (NUM_SUBCORES, SBUF, CHUNK, EMBED_DIM), jnp.float32),
        ]
        + [pltpu.VMEM((CHUNK, EMBED_DIM), jnp.float32)] * NBUF
        + [pltpu.SemaphoreType.DMA] * NBUF
        + [pltpu.SemaphoreType.DMA] * (2 * SBUF),
    )
    def body(x_hbm, seg_hbm, tab_hbm, out_hbm, tab_v, idx_all, xsh,
             *bufs_sems):
        xbufs = bufs_sems[:NBUF]
        isems = bufs_sems[NBUF:2 * NBUF]
        csems = bufs_sems[2 * NBUF:2 * NBUF + SBUF]
        osems = bufs_sems[2 * NBUF + SBUF:2 * NBUF + 2 * SBUF]
        cid = lax.axis_index("c")
        sid = lax.axis_index("s")
        wid = sid * NUM_CORES + cid
        base = wid * TOKENS_PER_WORKER

        pltpu.sync_copy(tab_hbm, tab_v)
        pltpu.sync_copy(seg_hbm.at[pl.ds(base, TOKENS_PER_WORKER)], idx_all)

        def issue_load(b, c):
            pltpu.async_copy(
                x_hbm.at[pl.ds(base + c * CHUNK, CHUNK)], xbufs[b], isems[b])

        def wait_load(b):
            pltpu.make_async_copy(
                x_hbm.at[pl.ds(0, CHUNK)], xbufs[b], isems[b]).wait()

        def issue_crossbar(b, s):
            pltpu.async_copy(xbufs[b], xsh.at[sid, s], csems[s])

        def wait_crossbar(s):
            pltpu.make_async_copy(xbufs[0], xsh.at[sid, s], csems[s]).wait()

        def issue_store(s, c):
            pltpu.async_copy(
                xsh.at[sid, s], out_hbm.at[pl.ds(base + c * CHUNK, CHUNK)],
                osems[s])

        def wait_store(s):
            pltpu.make_async_copy(
                xsh.at[sid, s], out_hbm.at[pl.ds(0, CHUNK)], osems[s]).wait()

        def compute(b, c):
            xbuf = xbufs[b]
            for g in range(CHUNK // LANES):
                segs = idx_all[pl.ds(c * CHUNK + g * LANES, LANES)]
                for t16 in range(LANES):
                    s = segs[t16]
                    t = g * LANES + t16

                    @plsc.parallel_loop(0, EMBED_DIM, LANES, unroll=8)
                    def pos(j):
                        plsc.addupdate(
                            xbuf.at[t, pl.ds(j, LANES)],
                            tab_v[s, pl.ds(j, LANES)],
                        )

        for b in range(LEAD):
            issue_load(b, b)

        def ring(i, carry):
            c0 = NBUF * i
            for b in range(NBUF):
                c = c0 + b
                s = b % SBUF
                sp = (b - 1) % SBUF
                nb = (b + LEAD) % NBUF
                wait_load(b)

                @pl.when(c + LEAD < NUM_CHUNKS)
                def _():
                    issue_load(nb, c + LEAD)

                compute(b, c)

                @pl.when(c >= 1)
                def _():
                    wait_crossbar(sp)
                    issue_store(sp, c - 1)

                @pl.when(c >= SBUF)
                def _():
                    wait_store(s)

                issue_crossbar(b, s)

            return carry

        lax.fori_loop(0, NUM_CHUNKS // NBUF, ring, 0)

        last = (NUM_CHUNKS - 1) % SBUF
        wait_crossbar(last)
        issue_store(last, NUM_CHUNKS - 1)
        for s in range(SBUF):
            wait_store(s)

    return body(x2d, seg, table)


def kernel(x, segment_ids, table):
    x2d = x.reshape(N_TOKENS, EMBED_DIM)
    seg = segment_ids.reshape(N_TOKENS).astype(jnp.int32)
    out = _seg_encode(x2d, seg, table)
    return out.reshape(BATCH, SEQ_LEN, EMBED_DIM)


# speedup vs baseline: 1.3811x; 1.0035x over previous
import functools

import jax
import jax.numpy as jnp
from jax import lax
from jax.experimental import pallas as pl
from jax.experimental.pallas import tpu as pltpu
from jax.experimental.pallas import tpu_sc as plsc

BATCH = 4
SEQ_LEN = 8192
EMBED_DIM = 1024
NUM_SEGMENTS = 16
LANES = 16

N_TOKENS = BATCH * SEQ_LEN
NUM_CORES = 2
NUM_SUBCORES = 16
NUM_WORKERS = NUM_CORES * NUM_SUBCORES
TOKENS_PER_WORKER = N_TOKENS // NUM_WORKERS
CHUNK = 16
NUM_CHUNKS = TOKENS_PER_WORKER // CHUNK
NBUF = 4
SBUF = 2
LEAD = 2


@jax.jit
def _seg_encode(x2d, seg, table):
    mesh = plsc.VectorSubcoreMesh(core_axis_name="c", subcore_axis_name="s")

    @functools.partial(
        pl.kernel,
        mesh=mesh,
        out_type=jax.ShapeDtypeStruct((N_TOKENS, EMBED_DIM), jnp.float32),
        scratch_types=[
            pltpu.VMEM((NUM_SEGMENTS, EMBED_DIM), jnp.float32),
            pltpu.VMEM((TOKENS_PER_WORKER,), jnp.int32),
            pltpu.VMEM_SHARED(
                (NUM_SUBCORES, SBUF, CHUNK, EMBED_DIM), jnp.float32),
        ]
        + [pltpu.VMEM((CHUNK, EMBED_DIM), jnp.float32)] * NBUF
        + [pltpu.SemaphoreType.DMA] * NBUF
        + [pltpu.SemaphoreType.DMA] * (2 * SBUF),
    )
    def body(x_hbm, seg_hbm, tab_hbm, out_hbm, tab_v, idx_all, xsh,
             *bufs_sems):
        xbufs = bufs_sems[:NBUF]
        isems = bufs_sems[NBUF:2 * NBUF]
        csems = bufs_sems[2 * NBUF:2 * NBUF + SBUF]
        osems = bufs_sems[2 * NBUF + SBUF:2 * NBUF + 2 * SBUF]
        cid = lax.axis_index("c")
        sid = lax.axis_index("s")
        wid = sid * NUM_CORES + cid
        base = wid * TOKENS_PER_WORKER

        def issue_load(b, c):
            pltpu.async_copy(
                x_hbm.at[pl.ds(base + c * CHUNK, CHUNK)], xbufs[b], isems[b])

        def wait_load(b):
            pltpu.make_async_copy(
                x_hbm.at[pl.ds(0, CHUNK)], xbufs[b], isems[b]).wait()

        def issue_crossbar(b, s):
            pltpu.async_copy(xbufs[b], xsh.at[sid, s], csems[s])

        def wait_crossbar(s):
            pltpu.make_async_copy(xbufs[0], xsh.at[sid, s], csems[s]).wait()

        def issue_store(s, c):
            pltpu.async_copy(
                xsh.at[sid, s], out_hbm.at[pl.ds(base + c * CHUNK, CHUNK)],
                osems[s])

        def wait_store(s):
            pltpu.make_async_copy(
                xsh.at[sid, s], out_hbm.at[pl.ds(0, CHUNK)], osems[s]).wait()

        def compute(b, c):
            xbuf = xbufs[b]
            for g in range(CHUNK // LANES):
                segs = idx_all[pl.ds(c * CHUNK + g * LANES, LANES)]
                for t16 in range(LANES):
                    s = segs[t16]
                    t = g * LANES + t16

                    @plsc.parallel_loop(0, EMBED_DIM, LANES, unroll=8)
                    def pos(j):
                        plsc.addupdate(
                            xbuf.at[t, pl.ds(j, LANES)],
                            tab_v[s, pl.ds(j, LANES)],
                        )

        for b in range(LEAD):
            issue_load(b, b)

        pltpu.sync_copy(tab_hbm, tab_v)
        pltpu.sync_copy(seg_hbm.at[pl.ds(base, TOKENS_PER_WORKER)], idx_all)

        def ring(i, carry):
            c0 = NBUF * i
            for b in range(NBUF):
                c = c0 + b
                s = b % SBUF
                sp = (b - 1) % SBUF
                nb = (b + LEAD) % NBUF
                wait_load(b)

                @pl.when(c + LEAD < NUM_CHUNKS)
                def _():
                    issue_load(nb, c + LEAD)

                compute(b, c)

                @pl.when(c >= 1)
                def _():
                    wait_crossbar(sp)
                    issue_store(sp, c - 1)

                @pl.when(c >= SBUF)
                def _():
                    wait_store(s)

                issue_crossbar(b, s)

            return carry

        lax.fori_loop(0, NUM_CHUNKS // NBUF, ring, 0)

        last = (NUM_CHUNKS - 1) % SBUF
        wait_crossbar(last)
        issue_store(last, NUM_CHUNKS - 1)
        for s in range(SBUF):
            wait_store(s)

    return body(x2d, seg, table)


def kernel(x, segment_ids, table):
    x2d = x.reshape(N_TOKENS, EMBED_DIM)
    seg = segment_ids.reshape(N_TOKENS).astype(jnp.int32)
    out = _seg_encode(x2d, seg, table)
    return out.reshape(BATCH, SEQ_LEN, EMBED_DIM)
